# baseline (speedup 1.0000x reference)
import functools

import jax
import jax.numpy as jnp
from jax import lax
from jax.experimental import pallas as pl
from jax.experimental.pallas import tpu as pltpu
from jax.experimental.pallas import tpu_sc as plsc

_NC = 2
_NS = 16
_NW = _NC * _NS
_L = 16


def kernel(inputs, reverse_p_array):
    b, c, h, w = inputs.shape
    n = h * w
    rows = b * c
    rows_per = rows // _NW
    ch = 64
    n_chunks = rows_per // ch
    vecs = ch * (n // _L)

    x = inputs.reshape(rows * n)
    perm = reverse_p_array.astype(jnp.int32)

    mesh = plsc.VectorSubcoreMesh(core_axis_name="c", subcore_axis_name="s")

    @functools.partial(
        pl.kernel,
        mesh=mesh,
        compiler_params=pltpu.CompilerParams(needs_layout_passes=False),
        out_type=jax.ShapeDtypeStruct((rows * n,), jnp.float32),
        scratch_types=[
            pltpu.VMEM((n,), jnp.int32),
            pltpu.VMEM((ch * n,), jnp.int32),
            pltpu.VMEM((ch * n,), jnp.float32),
            pltpu.VMEM((ch * n,), jnp.float32),
            pltpu.SemaphoreType.DMA,
        ],
    )
    def run(x_hbm, perm_hbm, out_hbm, perm_v, idx_v, in_v, out_v, sem):
        wid = lax.axis_index("s") * _NC + lax.axis_index("c")
        pltpu.sync_copy(perm_hbm, perm_v)

        def build(i, carry):
            r = i // (n // _L)
            g = i % (n // _L)
            pv = perm_v[pl.ds(g * _L, _L)]
            idx_v[pl.ds(i * _L, _L)] = pv + r * n
            return carry

        lax.fori_loop(0, vecs, build, 0)

        base = wid * rows_per * n

        def chunk(k, carry):
            off = base + k * ch * n
            pltpu.async_copy(x_hbm.at[pl.ds(off, ch * n)], in_v, sem).wait()

            def body(i, c2):
                iv = idx_v[pl.ds(i * _L, _L)]
                out_v[pl.ds(i * _L, _L)] = plsc.load_gather(in_v, [iv])
                return c2

            lax.fori_loop(0, vecs, body, 0)
            pltpu.async_copy(out_v, out_hbm.at[pl.ds(off, ch * n)], sem).wait()
            return carry

        lax.fori_loop(0, n_chunks, chunk, 0)

    y = run(x, perm)
    return y.reshape(b, c, h, w)


# speedup vs baseline: 1.1286x; 1.1286x over previous
import functools

import jax
import jax.numpy as jnp
from jax import lax
from jax.experimental import pallas as pl
from jax.experimental.pallas import tpu as pltpu
from jax.experimental.pallas import tpu_sc as plsc

_NC = 2
_NS = 16
_NW = _NC * _NS
_L = 16


def kernel(inputs, reverse_p_array):
    b, c, h, w = inputs.shape
    n = h * w
    ng = n // _L
    rows = b * c
    rows_per = rows // _NW
    ch = 64
    n_chunks = rows_per // ch

    x = inputs.reshape(rows * n)
    perm = reverse_p_array.astype(jnp.int32)

    mesh = plsc.VectorSubcoreMesh(core_axis_name="c", subcore_axis_name="s")

    @functools.partial(
        pl.kernel,
        mesh=mesh,
        compiler_params=pltpu.CompilerParams(needs_layout_passes=False),
        out_type=jax.ShapeDtypeStruct((rows * n,), jnp.float32),
        scratch_types=[
            pltpu.VMEM((n,), jnp.int32),
            pltpu.VMEM((ch * n,), jnp.float32),
            pltpu.VMEM((ch * n,), jnp.float32),
            pltpu.VMEM((ch * n,), jnp.float32),
            pltpu.VMEM((ch * n,), jnp.float32),
            pltpu.SemaphoreType.DMA,
            pltpu.SemaphoreType.DMA,
            pltpu.SemaphoreType.DMA,
            pltpu.SemaphoreType.DMA,
        ],
    )
    def run(x_hbm, perm_hbm, out_hbm, perm_v, in_v0, in_v1, out_v0, out_v1,
            sem_i0, sem_i1, sem_o0, sem_o1):
        wid = lax.axis_index("s") * _NC + lax.axis_index("c")
        pltpu.sync_copy(perm_hbm, perm_v)

        pvecs = [perm_v[pl.ds(g * _L, _L)] for g in range(ng)]

        base = wid * rows_per * n
        in_bufs = (in_v0, in_v1)
        out_bufs = (out_v0, out_v1)
        sems_i = (sem_i0, sem_i1)
        sems_o = (sem_o0, sem_o1)

        def in_copy(k, slot):
            off = base + k * ch * n
            return pltpu.make_async_copy(
                x_hbm.at[pl.ds(off, ch * n)], in_bufs[slot], sems_i[slot])

        def out_copy(k, slot):
            off = base + k * ch * n
            return pltpu.make_async_copy(
                out_bufs[slot], out_hbm.at[pl.ds(off, ch * n)], sems_o[slot])

        def permute_chunk(slot):
            src = in_bufs[slot]
            dst = out_bufs[slot]

            def row(r, carry):
                rb = r * n
                for g in range(ng):
                    iv = pvecs[g] + rb
                    dst[pl.ds(rb + g * _L, _L)] = plsc.load_gather(src, [iv])
                return carry

            lax.fori_loop(0, ch, row, 0, unroll=2)

        in_copy(0, 0).start()

        def pair(p, carry):
            for slot in range(2):
                k = p * 2 + slot

                @pl.when(k + 1 < n_chunks)
                def _():
                    in_copy(k + 1, 1 - slot).start()

                in_copy(k, slot).wait()

                @pl.when(k >= 2)
                def _():
                    out_copy(k - 2, slot).wait()

                permute_chunk(slot)
                out_copy(k, slot).start()
            return carry

        lax.fori_loop(0, n_chunks // 2, pair, 0)
        out_copy(n_chunks - 2, 0).wait()
        out_copy(n_chunks - 1, 1).wait()

    y = run(x, perm)
    return y.reshape(b, c, h, w)
